# baseline (speedup 1.0000x reference)
import functools

import jax
import jax.numpy as jnp
from jax import lax
from jax.experimental import pallas as pl
from jax.experimental.pallas import tpu as pltpu
from jax.experimental.pallas import tpu_sc as plsc


def _make_sc_gather(feat, nw, ch, cw, kbuf):
    perw = ch * cw
    outer = ch // kbuf
    rows = kbuf * cw
    mesh = plsc.VectorSubcoreMesh(core_axis_name="c", subcore_axis_name="s")
    ncores = plsc.get_sparse_core_info().num_cores

    def body(x_hbm, idx_hbm, out_hbm, idx_v, rows0_v, rows1_v, sem_g, sem_w):
        wid = lax.axis_index("s") * ncores + lax.axis_index("c")
        pltpu.sync_copy(idx_hbm.at[wid], idx_v)
        bufs = (rows0_v, rows1_v)

        def step(o, carry):
            def run(buf):
                copies = []
                for k in range(kbuf):
                    copies.append(pltpu.async_copy(
                        x_hbm.at[idx_v.at[o * kbuf + k]],
                        buf.at[pl.ds(k * cw, cw)],
                        sem_g,
                    ))
                for cp in copies:
                    cp.wait()
                dst = out_hbm.at[pl.ds(wid * perw + o * rows, rows)]
                pltpu.async_copy(buf, dst, sem_w)

                @pl.when(o > 0)
                def _():
                    pltpu.make_async_copy(buf, dst, sem_w).wait()

            @pl.when(lax.rem(o, 2) == 0)
            def _():
                run(bufs[0])

            @pl.when(lax.rem(o, 2) == 1)
            def _():
                run(bufs[1])

            return carry

        lax.fori_loop(0, outer, step, 0)
        pltpu.make_async_copy(
            rows0_v, out_hbm.at[pl.ds(wid * perw, rows)], sem_w).wait()

    return pl.kernel(
        body,
        mesh=mesh,
        out_type=jax.ShapeDtypeStruct((nw * perw, feat), jnp.float32),
        scratch_types=[
            pltpu.VMEM((ch, cw), jnp.int32),
            pltpu.VMEM((rows, feat), jnp.float32),
            pltpu.VMEM((rows, feat), jnp.float32),
            pltpu.SemaphoreType.DMA,
            pltpu.SemaphoreType.DMA,
        ],
    )


def _pick_chunking(perw, row_bytes):
    best = None
    for cw in range(128, 0, -1):
        if perw % cw:
            continue
        ch = perw // cw
        for kbuf in (8, 6, 5, 4, 3, 2, 1):
            if ch % kbuf or (cw * kbuf) % 8:
                continue
            if 2 * cw * kbuf * row_bytes > 420 * 1024:
                continue
            if best is None or (cw * kbuf, kbuf) > (best[0] * best[2], best[2]):
                best = (cw, ch, kbuf)
            break
    return best


def _lstm_gates(m_blk, hs, cs, wcat_ref, b_ref, hid):
    wdt = wcat_ref.dtype
    xx = jnp.concatenate([m_blk.astype(wdt), hs.astype(wdt)], axis=-1)
    gates = jnp.dot(xx, wcat_ref[:], preferred_element_type=jnp.float32) + b_ref[:]
    i = jax.nn.sigmoid(gates[:, :hid])
    f = jax.nn.sigmoid(gates[:, hid:2 * hid])
    g = jnp.tanh(gates[:, 2 * hid:3 * hid])
    o = jax.nn.sigmoid(gates[:, 3 * hid:])
    c_new = f * cs + i * g
    h_new = o * jnp.tanh(c_new)
    return c_new, h_new


def _layer_body(m_ref, x_ref, wcat_ref, b_ref, ws_ref, wn_ref, bo_ref,
                out_ref, hs_ref, cs_ref, *, steps, rows, hid, relu):
    t = pl.program_id(0)

    @pl.when(t == 0)
    def _():
        hs_ref[:, :] = jnp.zeros((rows, hid), jnp.float32)
        cs_ref[:, :] = jnp.zeros((rows, hid), jnp.float32)

    c_new, h_new = _lstm_gates(m_ref[0], hs_ref[:, :], cs_ref[:, :],
                               wcat_ref, b_ref, hid)
    cs_ref[:, :] = c_new
    hs_ref[:, :] = h_new

    @pl.when(t == steps - 1)
    def _():
        rst = (jnp.dot(x_ref[:, :], ws_ref[:], preferred_element_type=jnp.float32)
               + jnp.dot(h_new, wn_ref[:], preferred_element_type=jnp.float32)
               + bo_ref[:])
        out_ref[:, :] = jnp.maximum(rst, 0.0) if relu else rst


def _psum_body(m_ref, x_ref, wcat_ref, b_ref, out_ref, hs_ref, cs_ref,
               *, steps, rows, hid):
    t = pl.program_id(0)

    @pl.when(t == 0)
    def _():
        hs_ref[:, :] = jnp.zeros((rows, hid), jnp.float32)
        cs_ref[:, :] = jnp.zeros((rows, hid), jnp.float32)

    c_new, h_new = _lstm_gates(m_ref[0], hs_ref[:, :], cs_ref[:, :],
                               wcat_ref, b_ref, hid)
    cs_ref[:, :] = c_new
    hs_ref[:, :] = h_new

    @pl.when(t == steps - 1)
    def _():
        out_ref[0:1, :] = jnp.sum(h_new, axis=0, keepdims=True)
        out_ref[1:2, :] = jnp.sum(x_ref[:, :], axis=0, keepdims=True)


def _head_body(p_ref, ws_ref, wn_ref, bo_ref, w0_ref, b0_ref, w1_ref, b1_ref,
               w2_ref, b2_ref, wl_ref, bl_ref, out_ref, *, n_nodes, hid):
    inv_n = jnp.float32(1.0 / n_nodes)
    hm = jnp.sum(p_ref[:, 0, :], axis=0, keepdims=True) * inv_n
    xm = jnp.sum(p_ref[:, 1, :], axis=0, keepdims=True) * inv_n
    rst = (jnp.dot(xm, ws_ref[:], preferred_element_type=jnp.float32)
           + jnp.dot(hm, wn_ref[:], preferred_element_type=jnp.float32)
           + bo_ref[:])
    y = jnp.maximum(jnp.dot(rst, w0_ref[:], preferred_element_type=jnp.float32) + b0_ref[:], 0.0)
    y = jnp.maximum(jnp.dot(y, w1_ref[:], preferred_element_type=jnp.float32) + b1_ref[:], 0.0)
    y = jnp.dot(y, w2_ref[:], preferred_element_type=jnp.float32) + b2_ref[:]
    out_ref[:, :] = jnp.dot(y, wl_ref[:], preferred_element_type=jnp.float32) + bl_ref[:]


def _cparams():
    return pltpu.CompilerParams(
        dimension_semantics=("arbitrary",),
        vmem_limit_bytes=100 * 1024 * 1024,
    )


def _const_spec(shape):
    return pl.BlockSpec(shape, lambda t: tuple(0 for _ in shape))


def _lstm_in_specs(rows, hid):
    return [
        pl.BlockSpec((1, rows, hid), lambda t: (t, 0, 0)),
        _const_spec((rows, hid)),
        _const_spec((2 * hid, 4 * hid)),
        _const_spec((1, 4 * hid)),
    ]


def _make_layer_call(rows, steps, hid, relu):
    in_specs = _lstm_in_specs(rows, hid) + [
        _const_spec((hid, hid)),
        _const_spec((hid, hid)),
        _const_spec((1, hid)),
    ]
    return pl.pallas_call(
        functools.partial(_layer_body, steps=steps, rows=rows, hid=hid, relu=relu),
        grid=(steps,),
        in_specs=in_specs,
        out_specs=_const_spec((rows, hid)),
        out_shape=jax.ShapeDtypeStruct((rows, hid), jnp.float32),
        scratch_shapes=[
            pltpu.VMEM((rows, hid), jnp.float32),
            pltpu.VMEM((rows, hid), jnp.float32),
        ],
        compiler_params=_cparams(),
    )


def _make_psum_call(rows, steps, hid):
    return pl.pallas_call(
        functools.partial(_psum_body, steps=steps, rows=rows, hid=hid),
        grid=(steps,),
        in_specs=_lstm_in_specs(rows, hid),
        out_specs=_const_spec((2, hid)),
        out_shape=jax.ShapeDtypeStruct((2, hid), jnp.float32),
        scratch_shapes=[
            pltpu.VMEM((rows, hid), jnp.float32),
            pltpu.VMEM((rows, hid), jnp.float32),
        ],
        compiler_params=_cparams(),
    )


def _make_head_call(k, n, hid, nc):
    specs = [pl.BlockSpec((k, 2, hid), lambda: (0, 0, 0))]
    for shape in [(hid, hid), (hid, hid), (1, hid), (hid, hid), (1, hid),
                  (hid, hid), (1, hid), (hid, hid), (1, hid), (hid, nc), (1, nc)]:
        specs.append(pl.BlockSpec(shape, lambda *_, s=shape: tuple(0 for _ in s)))
    return pl.pallas_call(
        functools.partial(_head_body, n_nodes=n, hid=hid),
        in_specs=specs,
        out_specs=pl.BlockSpec((1, nc), lambda: (0, 0)),
        out_shape=jax.ShapeDtypeStruct((1, nc), jnp.float32),
    )


def kernel(h, edge_index, conv_params, mlp_params, lin_W, lin_b):
    n, d = h.shape
    e = edge_index.shape[1]
    deg = e // n
    hid = conv_params[0]['W_self'].shape[0]
    nc = lin_W.shape[0]

    info = plsc.get_sparse_core_info()
    nw = info.num_cores * info.num_subcores

    k = 1
    for cand in (5, 4, 2):
        if n % cand == 0 and (e // cand) % nw == 0 and (e // cand // nw) % 8 == 0:
            k = cand
            break
    nck = n // k
    perw = e // k // nw
    cw, ch, kbuf = _pick_chunking(perw, hid * 4)

    src = edge_index[0]
    src_cm = jnp.transpose(src.reshape(k, nck, deg), (0, 2, 1)).reshape(k, nw, ch, cw)

    sc_gather = _make_sc_gather(hid, nw, ch, cw, kbuf)
    layer_call = _make_layer_call(nck, deg, hid, relu=True)
    psum_call = _make_psum_call(nck, deg, hid)
    head_call = _make_head_call(k, n, hid, nc)

    def layer_weights(p):
        wcat = jnp.transpose(jnp.concatenate([p['W_ih'], p['W_hh']], axis=1))
        bias = (p['b_ih'] + p['b_hh']).reshape(1, -1)
        return (wcat.astype(jnp.bfloat16), bias, jnp.transpose(p['W_self']),
                jnp.transpose(p['W_neigh']), p['b'].reshape(1, -1))

    x = h
    for li in range(6):
        wcat, bias, ws, wn, bo = layer_weights(conv_params[li])
        ms = [sc_gather(x, src_cm[c]).reshape(deg, nck, hid) for c in range(k)]
        if li < 5:
            outs = [layer_call(ms[c], x[c * nck:(c + 1) * nck], wcat, bias,
                               ws, wn, bo) for c in range(k)]
            x = jnp.concatenate(outs, axis=0) if k > 1 else outs[0]
        else:
            parts = [psum_call(ms[c], x[c * nck:(c + 1) * nck], wcat, bias)
                     for c in range(k)]
            psum = jnp.stack(parts, axis=0)
            mlp = []
            for p in mlp_params:
                mlp.extend([jnp.transpose(p['W']), p['b'].reshape(1, -1)])
            return head_call(psum, ws, wn, bo, *mlp,
                             jnp.transpose(lin_W), lin_b.reshape(1, -1))
